# baseline (speedup 1.0000x reference)
import functools

import jax
import jax.numpy as jnp
from jax import lax
from jax.experimental import pallas as pl
from jax.experimental.pallas import tpu as pltpu
from jax.experimental.pallas import tpu_sc as plsc

_GT_POS_THR = 3.0
_GT_NEG_THR = 5.0
_IMG_W = 512.0
_IMG_H = 512.0
_UNMATCHED = -1
_IGNORE = -2

_ROW_BLOCK = 256
_LANES = 16


def _reproject(kpts, H):
    ones = jnp.ones(kpts.shape[:-1] + (1,), dtype=kpts.dtype)
    homo = jnp.concatenate([kpts, ones], axis=-1)
    proj = jnp.einsum('bij,bnj->bni', H, homo)
    z = proj[..., 2]
    zs = jnp.where(jnp.abs(z) > 1e-8, z, 1e-8)
    xy = proj[..., :2] / zs[..., None]
    mask = ((z > 1e-8)
            & (xy[..., 0] >= 0.0) & (xy[..., 0] < _IMG_W)
            & (xy[..., 1] >= 0.0) & (xy[..., 1] < _IMG_H))
    return xy, mask


def _dist_block_kernel(aa_ref, qx_ref, qy_ref, bb_ref, kx_ref, ky_ref,
                       md_ref, am_ref):
    aa = aa_ref[0, 0]
    qx = qx_ref[0, 0]
    qy = qy_ref[0, 0]
    bb = bb_ref[0, 0]
    kx = kx_ref[0, 0]
    ky = ky_ref[0, 0]
    ab = qx * kx + qy * ky
    dist = jnp.sqrt(jnp.maximum((aa + bb) - 2.0 * ab, 1e-12))
    m = jnp.min(dist, axis=1, keepdims=True)
    n = dist.shape[1]
    iota = lax.broadcasted_iota(jnp.int32, dist.shape, 1)
    am = jnp.min(jnp.where(dist == m, iota, n), axis=1, keepdims=True)
    md_ref[0, 0] = m
    am_ref[0, 0] = am


def _min_argmin(aa, qx, qy, bb, kx, ky):
    two, B, N, _ = qx.shape
    R = _ROW_BLOCK
    q_spec = pl.BlockSpec((1, 1, R, 1), lambda d, b, r: (d, b, r, 0))
    k_spec = pl.BlockSpec((1, 1, 1, N), lambda d, b, r: (d, b, 0, 0))
    o_spec = pl.BlockSpec((1, 1, R, 1), lambda d, b, r: (d, b, r, 0))
    md, am = pl.pallas_call(
        _dist_block_kernel,
        grid=(two, B, N // R),
        in_specs=[q_spec, q_spec, q_spec, k_spec, k_spec, k_spec],
        out_specs=[o_spec, o_spec],
        out_shape=[
            jax.ShapeDtypeStruct((two, B, N, 1), jnp.float32),
            jax.ShapeDtypeStruct((two, B, N, 1), jnp.int32),
        ],
        compiler_params=pltpu.CompilerParams(
            dimension_semantics=("parallel", "parallel", "parallel")),
    )(aa, qx, qy, bb, kx, ky)
    return md, am


def _epilogue_body(B, N,
                   am0_h, am1_h, md0_h, md1_h, msk_h, gm0_h, gm1_h,
                   am0_v, am1_v, md0_v, md1_v, msk_v, gm0_v, gm1_v):
    c = lax.axis_index("c")
    s = lax.axis_index("s")
    wid = s * 2 + c

    @pl.when(wid < B)
    def _():
        b = wid
        pltpu.sync_copy(am0_h.at[b], am0_v)
        pltpu.sync_copy(am1_h.at[b], am1_v)
        pltpu.sync_copy(md0_h.at[b], md0_v)
        pltpu.sync_copy(md1_h.at[b], md1_v)
        pltpu.sync_copy(msk_h.at[b], msk_v)
        lane = lax.iota(jnp.int32, _LANES)

        def body0(i, carry):
            sl = pl.ds(i * _LANES, _LANES)
            idx0 = am0_v[sl]
            back = plsc.load_gather(am1_v, [idx0])
            gi = lane + i * _LANES
            md0 = md0_v[sl]
            msk = msk_v[sl]
            g = jnp.where(back == gi, idx0, _UNMATCHED)
            g = jnp.where(msk == 0, _IGNORE, g)
            g = jnp.where(md0 > _GT_POS_THR, _IGNORE, g)
            g = jnp.where(md0 > _GT_NEG_THR, _UNMATCHED, g)
            gm0_v[sl] = g
            return carry

        lax.fori_loop(0, N // _LANES, body0, 0)

        def body1(j, carry):
            sl = pl.ds(j * _LANES, _LANES)
            ij = am1_v[sl]
            g0 = plsc.load_gather(gm0_v, [ij])
            gj = lane + j * _LANES
            md1 = md1_v[sl]
            base = jnp.where(md1 > _GT_NEG_THR, _UNMATCHED, _IGNORE)
            gm1_v[sl] = jnp.where(g0 == gj, ij, base)
            return carry

        lax.fori_loop(0, N // _LANES, body1, 0)
        pltpu.sync_copy(gm0_v, gm0_h.at[b])
        pltpu.sync_copy(gm1_v, gm1_h.at[b])


def _epilogue(am0, am1, md0, md1, msk):
    B, N = am0.shape
    mesh = plsc.VectorSubcoreMesh(core_axis_name="c", subcore_axis_name="s")
    body = functools.partial(_epilogue_body, B, N)
    f = pl.kernel(
        body,
        mesh=mesh,
        compiler_params=pltpu.CompilerParams(needs_layout_passes=False),
        out_type=[
            jax.ShapeDtypeStruct((B, N), jnp.int32),
            jax.ShapeDtypeStruct((B, N), jnp.int32),
        ],
        scratch_types=[
            pltpu.VMEM((N,), jnp.int32),
            pltpu.VMEM((N,), jnp.int32),
            pltpu.VMEM((N,), jnp.float32),
            pltpu.VMEM((N,), jnp.float32),
            pltpu.VMEM((N,), jnp.int32),
            pltpu.VMEM((N,), jnp.int32),
            pltpu.VMEM((N,), jnp.int32),
        ],
    )
    return f(am0, am1, md0, md1, msk)


def kernel(kpts0, kpts1, transformation):
    B, N0, _ = kpts0.shape
    N1 = kpts1.shape[1]
    transformation_inv = jnp.linalg.inv(transformation)
    kpts0_t, mask0 = _reproject(kpts0, transformation)
    kpts1_t, mask1 = _reproject(kpts1, transformation_inv)

    q = jnp.stack([kpts0_t, kpts1_t])
    k = jnp.stack([kpts1, kpts0])
    aa = jnp.sum(q * q, axis=-1)[..., None]
    bb = jnp.sum(k * k, axis=-1)[:, :, None, :]
    qb = q.astype(jnp.bfloat16).astype(jnp.float32)
    kb = k.astype(jnp.bfloat16).astype(jnp.float32)
    qx = qb[..., 0][..., None]
    qy = qb[..., 1][..., None]
    kx = kb[..., 0][:, :, None, :]
    ky = kb[..., 1][:, :, None, :]

    md, am = _min_argmin(aa, qx, qy, bb, kx, ky)
    min_dist0 = md[0, :, :, 0]
    min_dist1 = md[1, :, :, 0]
    am0 = am[0, :, :, 0]
    am1 = am[1, :, :, 0]

    gm0, gm1 = _epilogue(am0, am1, min_dist0, min_dist1,
                         mask0.astype(jnp.int32))
    return (gm0, gm1, min_dist0, min_dist1, kpts0, kpts1)


# speedup vs baseline: 1.6712x; 1.6712x over previous
import functools

import jax
import jax.numpy as jnp
from jax import lax
from jax.experimental import pallas as pl
from jax.experimental.pallas import tpu as pltpu
from jax.experimental.pallas import tpu_sc as plsc

_GT_POS_THR = 3.0
_GT_NEG_THR = 5.0
_IMG_W = 512.0
_IMG_H = 512.0
_UNMATCHED = -1
_IGNORE = -2

_ROW_BLOCK = 512
_LANES = 16


def _reproject(kpts, H):
    ones = jnp.ones(kpts.shape[:-1] + (1,), dtype=kpts.dtype)
    homo = jnp.concatenate([kpts, ones], axis=-1)
    proj = jnp.einsum('bij,bnj->bni', H, homo)
    z = proj[..., 2]
    zs = jnp.where(jnp.abs(z) > 1e-8, z, 1e-8)
    xy = proj[..., :2] / zs[..., None]
    mask = ((z > 1e-8)
            & (xy[..., 0] >= 0.0) & (xy[..., 0] < _IMG_W)
            & (xy[..., 1] >= 0.0) & (xy[..., 1] < _IMG_H))
    return xy, mask


def _dist_block_kernel(aa_ref, q_ref, bb_ref, k_ref, iota_ref, md_ref, am_ref):
    aa = aa_ref[0, 0]
    bb = bb_ref[0, 0]
    q2 = q_ref[0, 0]
    kk = k_ref[0, 0]
    iota = iota_ref[0, 0]
    ab2 = lax.dot_general(q2, kk, (((1,), (0,)), ((), ())),
                          preferred_element_type=jnp.float32)
    d2 = (aa + bb) - ab2
    m = jnp.maximum(jnp.min(d2, axis=1, keepdims=True), 1e-12)
    n = jnp.float32(d2.shape[1])
    amf = jnp.min(jnp.where(d2 <= m, iota, n), axis=1, keepdims=True)
    md_ref[0, 0] = jnp.sqrt(m)
    am_ref[0, 0] = amf.astype(jnp.int32)


def _min_argmin(aa, q2, bb, kt):
    two, B, N, _ = aa.shape
    R = _ROW_BLOCK
    iota = jnp.arange(N, dtype=jnp.float32).reshape(1, 1, 1, N)
    md, am = pl.pallas_call(
        _dist_block_kernel,
        grid=(two, B, N // R),
        in_specs=[
            pl.BlockSpec((1, 1, R, 1), lambda d, b, r: (d, b, r, 0)),
            pl.BlockSpec((1, 1, R, 2), lambda d, b, r: (d, b, r, 0)),
            pl.BlockSpec((1, 1, 1, N), lambda d, b, r: (d, b, 0, 0)),
            pl.BlockSpec((1, 1, 2, N), lambda d, b, r: (d, b, 0, 0)),
            pl.BlockSpec((1, 1, 1, N), lambda d, b, r: (0, 0, 0, 0)),
        ],
        out_specs=[
            pl.BlockSpec((1, 1, R, 1), lambda d, b, r: (d, b, r, 0)),
            pl.BlockSpec((1, 1, R, 1), lambda d, b, r: (d, b, r, 0)),
        ],
        out_shape=[
            jax.ShapeDtypeStruct((two, B, N, 1), jnp.float32),
            jax.ShapeDtypeStruct((two, B, N, 1), jnp.int32),
        ],
        compiler_params=pltpu.CompilerParams(
            dimension_semantics=("parallel", "parallel", "parallel")),
    )(aa, q2, bb, kt, iota)
    return md, am


def _epilogue_body(B, N,
                   am0_h, am1_h, md0_h, md1_h, msk_h, gm0_h, gm1_h,
                   am0_v, am1_v, md0_v, md1_v, msk_v, gm0_v, gm1_v):
    c = lax.axis_index("c")
    s = lax.axis_index("s")
    wid = s * 2 + c

    @pl.when(wid < B)
    def _():
        b = wid
        pltpu.sync_copy(am0_h.at[b], am0_v)
        pltpu.sync_copy(am1_h.at[b], am1_v)
        pltpu.sync_copy(md0_h.at[b], md0_v)
        pltpu.sync_copy(md1_h.at[b], md1_v)
        pltpu.sync_copy(msk_h.at[b], msk_v)
        lane = lax.iota(jnp.int32, _LANES)

        def body0(i, carry):
            sl = pl.ds(i * _LANES, _LANES)
            idx0 = am0_v[sl]
            back = plsc.load_gather(am1_v, [idx0])
            gi = lane + i * _LANES
            md0 = md0_v[sl]
            msk = msk_v[sl]
            g = jnp.where(back == gi, idx0, _UNMATCHED)
            g = jnp.where(msk == 0, _IGNORE, g)
            g = jnp.where(md0 > _GT_POS_THR, _IGNORE, g)
            g = jnp.where(md0 > _GT_NEG_THR, _UNMATCHED, g)
            gm0_v[sl] = g
            return carry

        lax.fori_loop(0, N // _LANES, body0, 0)

        def body1(j, carry):
            sl = pl.ds(j * _LANES, _LANES)
            ij = am1_v[sl]
            g0 = plsc.load_gather(gm0_v, [ij])
            gj = lane + j * _LANES
            md1 = md1_v[sl]
            base = jnp.where(md1 > _GT_NEG_THR, _UNMATCHED, _IGNORE)
            gm1_v[sl] = jnp.where(g0 == gj, ij, base)
            return carry

        lax.fori_loop(0, N // _LANES, body1, 0)
        pltpu.sync_copy(gm0_v, gm0_h.at[b])
        pltpu.sync_copy(gm1_v, gm1_h.at[b])


def _epilogue(am0, am1, md0, md1, msk):
    B, N = am0.shape
    mesh = plsc.VectorSubcoreMesh(core_axis_name="c", subcore_axis_name="s")
    body = functools.partial(_epilogue_body, B, N)
    f = pl.kernel(
        body,
        mesh=mesh,
        compiler_params=pltpu.CompilerParams(needs_layout_passes=False),
        out_type=[
            jax.ShapeDtypeStruct((B, N), jnp.int32),
            jax.ShapeDtypeStruct((B, N), jnp.int32),
        ],
        scratch_types=[
            pltpu.VMEM((N,), jnp.int32),
            pltpu.VMEM((N,), jnp.int32),
            pltpu.VMEM((N,), jnp.float32),
            pltpu.VMEM((N,), jnp.float32),
            pltpu.VMEM((N,), jnp.int32),
            pltpu.VMEM((N,), jnp.int32),
            pltpu.VMEM((N,), jnp.int32),
        ],
    )
    return f(am0, am1, md0, md1, msk)


def kernel(kpts0, kpts1, transformation):
    B, N0, _ = kpts0.shape
    N1 = kpts1.shape[1]
    transformation_inv = jnp.linalg.inv(transformation)
    kpts0_t, mask0 = _reproject(kpts0, transformation)
    kpts1_t, mask1 = _reproject(kpts1, transformation_inv)

    q = jnp.stack([kpts0_t, kpts1_t])
    k = jnp.stack([kpts1, kpts0])
    aa = jnp.sum(q * q, axis=-1)[..., None]
    bb = jnp.sum(k * k, axis=-1)[:, :, None, :]
    q2 = q.astype(jnp.bfloat16) * jnp.bfloat16(2.0)
    kt = jnp.swapaxes(k.astype(jnp.bfloat16), -1, -2)

    md, am = _min_argmin(aa, q2, bb, kt)
    min_dist0 = md[0, :, :, 0]
    min_dist1 = md[1, :, :, 0]
    am0 = am[0, :, :, 0]
    am1 = am[1, :, :, 0]

    gm0, gm1 = _epilogue(am0, am1, min_dist0, min_dist1,
                         mask0.astype(jnp.int32))
    return (gm0, gm1, min_dist0, min_dist1, kpts0, kpts1)
